# Initial kernel scaffold; baseline (speedup 1.0000x reference)
import jax
import jax.numpy as jnp
from jax.experimental import pallas as pl


def kernel(pred_coords, true_coords, pred_atoms, true_atoms, pred_charges, true_charges, pred_bonds, true_bonds, batch, bond_aggregation_index, weights):
    raise NotImplementedError("write your pallas kernel here")


# speedup vs baseline: 9.4662x; 9.4662x over previous
import functools

import jax
import jax.numpy as jnp
from jax import lax
from jax.experimental import pallas as pl
from jax.experimental.pallas import tpu as pltpu
from jax.experimental.pallas import tpu_sc as plsc

_N = 100000
_E = 1600000
_B = 2048
_AC = 16
_CC = 6
_BC = 5

_LAA = 8192
_LAB = 6400
_NP = 131072
_NROWS = _NP // 128
_EROWS_PAD = 12544
_EP = _EROWS_PAD * 128

_NC = 2
_NS = 16
_NW = _NC * _NS

_BROWS_W = _EROWS_PAD // _NW
_BBLK = 56
_BNBLK = _BROWS_W // _BBLK
_AROWS_W = _NROWS // _NW
_TILE_N = _NP // _NS


def _atom_body(pc_ref, tc_ref, pa_ref, ta_ref, pch_ref, tch_ref,
               regr_ref, ace_ref, cce_ref):
    blk = pl.program_id(0)
    lanes = lax.broadcasted_iota(jnp.int32, (1, _LAA), 1) + blk * _LAA
    mask = lanes < _N

    d = pc_ref[...] - tc_ref[...]
    regr = jnp.sum(d * d, axis=0, keepdims=True) * (1.0 / 3.0)
    regr_ref[...] = jnp.where(mask, regr, 0.0)[None]

    pa = pa_ref[...]
    ta = ta_ref[0]
    m = jnp.max(pa, axis=0, keepdims=True)
    lse = jnp.log(jnp.sum(jnp.exp(pa - m), axis=0, keepdims=True)) + m
    onehot = lax.broadcasted_iota(jnp.int32, (_AC, _LAA), 0) == ta
    tgt = jnp.sum(jnp.where(onehot, pa, 0.0), axis=0, keepdims=True)
    ace_ref[...] = jnp.where(mask, lse - tgt, 0.0)[None]

    pch = pch_ref[...]
    tch = tch_ref[0]
    m2 = jnp.max(pch, axis=0, keepdims=True)
    lse2 = jnp.log(jnp.sum(jnp.exp(pch - m2), axis=0, keepdims=True)) + m2
    onehot2 = lax.broadcasted_iota(jnp.int32, (_CC, _LAA), 0) == tch
    tgt2 = jnp.sum(jnp.where(onehot2, pch, 0.0), axis=0, keepdims=True)
    cce_ref[...] = jnp.where(mask, lse2 - tgt2, 0.0)[None]


def _atom_values(pc_t, tc_t, pa_t, ta3, pch_t, tch3):
    nblk = _NP // _LAA
    out_sds = jax.ShapeDtypeStruct((nblk, 1, _LAA), jnp.float32)
    return pl.pallas_call(
        _atom_body,
        grid=(nblk,),
        in_specs=[
            pl.BlockSpec((3, _LAA), lambda i: (0, i)),
            pl.BlockSpec((3, _LAA), lambda i: (0, i)),
            pl.BlockSpec((_AC, _LAA), lambda i: (0, i)),
            pl.BlockSpec((1, 1, _LAA), lambda i: (i, 0, 0)),
            pl.BlockSpec((_CC, _LAA), lambda i: (0, i)),
            pl.BlockSpec((1, 1, _LAA), lambda i: (i, 0, 0)),
        ],
        out_specs=[
            pl.BlockSpec((1, 1, _LAA), lambda i: (i, 0, 0)),
            pl.BlockSpec((1, 1, _LAA), lambda i: (i, 0, 0)),
            pl.BlockSpec((1, 1, _LAA), lambda i: (i, 0, 0)),
        ],
        out_shape=[out_sds, out_sds, out_sds],
    )(pc_t, tc_t, pa_t, ta3, pch_t, tch3)


def _bond_body(pb_ref, tb_ref, ce_ref):
    pb = pb_ref[...]
    tb = tb_ref[0]
    m = jnp.max(pb, axis=0, keepdims=True)
    lse = jnp.log(jnp.sum(jnp.exp(pb - m), axis=0, keepdims=True)) + m
    onehot = lax.broadcasted_iota(jnp.int32, (_BC, _LAB), 0) == tb
    tgt = jnp.sum(jnp.where(onehot, pb, 0.0), axis=0, keepdims=True)
    ce_ref[...] = (lse - tgt)[None]


def _bond_ce(pb_t, tb3):
    nblk = _E // _LAB
    return pl.pallas_call(
        _bond_body,
        grid=(nblk,),
        in_specs=[
            pl.BlockSpec((_BC, _LAB), lambda i: (0, i)),
            pl.BlockSpec((1, 1, _LAB), lambda i: (i, 0, 0)),
        ],
        out_specs=pl.BlockSpec((1, 1, _LAB), lambda i: (i, 0, 0)),
        out_shape=jax.ShapeDtypeStruct((nblk, 1, _LAB), jnp.float32),
    )(pb_t, tb3)


def _zero_vmem(ref, nwords):
    z = jnp.zeros((16,), jnp.float32)

    def body(i, _):
        ref[pl.ds(i * 16, 16)] = z
        return 0

    lax.fori_loop(0, nwords // 16, body, 0)


def _bond_scatter_body(ce_hbm, idx_hbm, s0_hbm, s1_hbm, c0_hbm, c1_hbm,
                       idx_v, val_v, ones_v, zero_v, out_v,
                       acc_s, cnt_s, sem):
    ci = lax.axis_index("c")
    si = lax.axis_index("s")
    wid = si * _NC + ci

    one = jnp.ones((16,), jnp.float32)
    for v in range(8):
        ones_v[pl.ds(v * 16, 16)] = one
    _zero_vmem(zero_v, _TILE_N)
    pltpu.sync_copy(zero_v, acc_s.at[pl.ds(si * _TILE_N, _TILE_N)])
    pltpu.sync_copy(zero_v, cnt_s.at[pl.ds(si * _TILE_N, _TILE_N)])
    plsc.subcore_barrier()

    def blk_body(bi, _):
        base = wid * _BROWS_W + bi * _BBLK
        pltpu.sync_copy(idx_hbm.at[pl.ds(base, _BBLK)], idx_v)
        pltpu.sync_copy(ce_hbm.at[pl.ds(base, _BBLK)], val_v)
        for g in range(0, _BBLK, 7):
            descs = []
            for j in range(g, g + 7):
                descs.append(pltpu.async_copy(
                    val_v.at[j], acc_s.at[idx_v.at[j]], sem, add=True))
                descs.append(pltpu.async_copy(
                    ones_v, cnt_s.at[idx_v.at[j]], sem, add=True))
            for d in descs:
                d.wait()
        return 0

    lax.fori_loop(0, _BNBLK, blk_body, 0)
    plsc.subcore_barrier()

    sl = pl.ds(si * _TILE_N, _TILE_N)
    outs = [(s0_hbm, c0_hbm), (s1_hbm, c1_hbm)]
    for c in range(_NC):
        @pl.when(ci == c)
        def _():
            pltpu.sync_copy(acc_s.at[sl], out_v)
            pltpu.sync_copy(out_v, outs[c][0].at[sl])
            pltpu.sync_copy(cnt_s.at[sl], out_v)
            pltpu.sync_copy(out_v, outs[c][1].at[sl])


def _bond_scatter(ce2, idx2):
    mesh = plsc.VectorSubcoreMesh(core_axis_name="c", subcore_axis_name="s",
                                  num_cores=_NC, num_subcores=_NS)
    sds = jax.ShapeDtypeStruct((_NP,), jnp.float32)
    f = pl.kernel(
        _bond_scatter_body,
        out_type=[sds, sds, sds, sds],
        mesh=mesh,
        scratch_types=[
            pltpu.VMEM((_BBLK, 128), jnp.int32),
            pltpu.VMEM((_BBLK, 128), jnp.float32),
            pltpu.VMEM((128,), jnp.float32),
            pltpu.VMEM((_TILE_N,), jnp.float32),
            pltpu.VMEM((_TILE_N,), jnp.float32),
            pltpu.VMEM_SHARED((_NP,), jnp.float32),
            pltpu.VMEM_SHARED((_NP,), jnp.float32),
            pltpu.SemaphoreType.DMA,
        ],
    )
    return f(ce2, idx2)


def _batch_scatter_body(batch_hbm, regr_hbm, ace_hbm, cce_hbm,
                        s0_hbm, s1_hbm, c0_hbm, c1_hbm, gpart_hbm,
                        idx_v, r_v, a_v, c_v, s0_v, s1_v, c0_v, c1_v,
                        b_v, ones_v, zero_v, out_v,
                        g0, g1, g2, g3, g4, sem):
    ci = lax.axis_index("c")
    si = lax.axis_index("s")
    wid = si * _NC + ci
    base = wid * _AROWS_W

    grefs = [g0, g1, g2, g3, g4]
    _zero_vmem(zero_v, _B)
    for k in range(5):
        @pl.when(si == k)
        def _():
            pltpu.sync_copy(zero_v, grefs[k])
    plsc.subcore_barrier()

    rows = pl.ds(base, _AROWS_W)
    pltpu.sync_copy(batch_hbm.at[rows], idx_v)
    pltpu.sync_copy(regr_hbm.at[rows], r_v)
    pltpu.sync_copy(ace_hbm.at[rows], a_v)
    pltpu.sync_copy(cce_hbm.at[rows], c_v)
    pltpu.sync_copy(s0_hbm.at[rows], s0_v)
    pltpu.sync_copy(s1_hbm.at[rows], s1_v)
    pltpu.sync_copy(c0_hbm.at[rows], c0_v)
    pltpu.sync_copy(c1_hbm.at[rows], c1_v)

    iota16 = lax.iota(jnp.int32, 16)
    for j in range(_AROWS_W):
        for v in range(8):
            slv = pl.ds(v * 16, 16)
            s = s0_v[j, slv] + s1_v[j, slv]
            cnt = c0_v[j, slv] + c1_v[j, slv]
            b = (0.5 * s) / jnp.maximum(cnt, 1.0)
            gatom = iota16 + ((base + j) * 128 + v * 16)
            valid = gatom < _N
            b_v[j, slv] = jnp.where(valid, b, 0.0)
            ones_v[j, slv] = jnp.where(valid, 1.0, 0.0)

    vals = [r_v, a_v, c_v, b_v, ones_v]
    for g in range(0, _AROWS_W, 4):
        descs = []
        for j in range(g, g + 4):
            for k in range(5):
                descs.append(pltpu.async_copy(
                    vals[k].at[j], grefs[k].at[idx_v.at[j]], sem, add=True))
        for d in descs:
            d.wait()
    plsc.subcore_barrier()

    for c in range(_NC):
        for k in range(5):
            @pl.when((si == k) & (ci == c))
            def _():
                pltpu.sync_copy(grefs[k], out_v)
                pltpu.sync_copy(out_v,
                                gpart_hbm.at[pl.ds((c * 5 + k) * _B, _B)])


def _batch_scatter(batch2, regr2, ace2, cce2, s0, s1, c0, c1):
    mesh = plsc.VectorSubcoreMesh(core_axis_name="c", subcore_axis_name="s",
                                  num_cores=_NC, num_subcores=_NS)
    f = pl.kernel(
        _batch_scatter_body,
        out_type=jax.ShapeDtypeStruct((_NC * 5 * _B,), jnp.float32),
        mesh=mesh,
        scratch_types=[
            pltpu.VMEM((_AROWS_W, 128), jnp.int32),
            pltpu.VMEM((_AROWS_W, 128), jnp.float32),
            pltpu.VMEM((_AROWS_W, 128), jnp.float32),
            pltpu.VMEM((_AROWS_W, 128), jnp.float32),
            pltpu.VMEM((_AROWS_W, 128), jnp.float32),
            pltpu.VMEM((_AROWS_W, 128), jnp.float32),
            pltpu.VMEM((_AROWS_W, 128), jnp.float32),
            pltpu.VMEM((_AROWS_W, 128), jnp.float32),
            pltpu.VMEM((_AROWS_W, 128), jnp.float32),
            pltpu.VMEM((_AROWS_W, 128), jnp.float32),
            pltpu.VMEM((_B,), jnp.float32),
            pltpu.VMEM((_B,), jnp.float32),
            pltpu.VMEM_SHARED((_B,), jnp.float32),
            pltpu.VMEM_SHARED((_B,), jnp.float32),
            pltpu.VMEM_SHARED((_B,), jnp.float32),
            pltpu.VMEM_SHARED((_B,), jnp.float32),
            pltpu.VMEM_SHARED((_B,), jnp.float32),
            pltpu.SemaphoreType.DMA,
        ],
    )
    return f(batch2, regr2, ace2, cce2, s0, s1, c0, c1)


def _final_body(gpart_ref, w_ref, out_ref):
    gp = gpart_ref[0] + gpart_ref[1]
    sums = gp[:4, :]
    cnt = gp[4:5, :]
    mean = sums / jnp.maximum(cnt, 1.0)
    w = w_ref[...]
    masked = jnp.where(jnp.isnan(mean), 0.0, mean * w)
    out_ref[...] = jnp.sum(masked, axis=1, keepdims=True)


def _final(gpart, w2):
    return pl.pallas_call(
        _final_body,
        out_shape=jax.ShapeDtypeStruct((4, 1), jnp.float32),
    )(gpart, w2)


@jax.jit
def kernel(pred_coords, true_coords, pred_atoms, true_atoms, pred_charges,
           true_charges, pred_bonds, true_bonds, batch,
           bond_aggregation_index, weights):
    padn = ((0, _NP - _N), (0, 0))
    pc_t = jnp.pad(pred_coords, padn).T
    tc_t = jnp.pad(true_coords, padn).T
    pa_t = jnp.pad(pred_atoms, padn).T
    pch_t = jnp.pad(pred_charges, padn).T
    ta3 = jnp.pad(true_atoms, (0, _NP - _N)).reshape(_NP // _LAA, 1, _LAA)
    tch3 = jnp.pad(true_charges, (0, _NP - _N)).reshape(_NP // _LAA, 1, _LAA)

    regr, ace, cce = _atom_values(pc_t, tc_t, pa_t, ta3, pch_t, tch3)

    pb_t = pred_bonds.T
    tb3 = true_bonds.reshape(_E // _LAB, 1, _LAB)
    ce = _bond_ce(pb_t, tb3)

    ce2 = jnp.pad(ce.reshape(_E), (0, _EP - _E)).reshape(_EROWS_PAD, 128)
    idx2 = jnp.pad(bond_aggregation_index, (0, _EP - _E),
                   constant_values=_N).reshape(_EROWS_PAD, 128)
    s0, s1, c0, c1 = _bond_scatter(ce2, idx2)

    batch2 = jnp.pad(batch, (0, _NP - _N)).reshape(_NROWS, 128)
    gpart = _batch_scatter(batch2, regr.reshape(_NROWS, 128),
                           ace.reshape(_NROWS, 128), cce.reshape(_NROWS, 128),
                           s0.reshape(_NROWS, 128), s1.reshape(_NROWS, 128),
                           c0.reshape(_NROWS, 128), c1.reshape(_NROWS, 128))

    out = _final(gpart.reshape(_NC, 5, _B), weights.reshape(1, _B))
    return out.reshape(4)
